# baseline (speedup 1.0000x reference)
import jax
import jax.numpy as jnp
import numpy as np
from jax.experimental import pallas as pl

_NUM_CODES = 8192
_CODE_DIM = 32
_BT = 256
_TW = 2048


def _vq_body(z_ref, emb_ref, zq_ref, codes_ref, loss_ref, avg_ref,
             ent_ref, nent_ref):
    i = pl.program_id(0)
    nb = pl.num_programs(0)
    z = z_ref[...]
    emb = emb_ref[...]
    z2 = jnp.sum(z * z, axis=1, keepdims=True)
    e2 = jnp.sum(emb * emb, axis=1, keepdims=True).T
    m = jax.lax.dot_general(z, emb, (((1,), (1,)), ((), ())),
                            preferred_element_type=jnp.float32)
    dist = (z2 - 2.0 * m) + e2

    acc_v = jnp.full((_BT,), jnp.inf, jnp.bfloat16)
    acc_i = jnp.zeros((_BT,), jnp.int32)
    for t in range(_NUM_CODES // _TW):
        dt = dist[:, t * _TW:(t + 1) * _TW]
        mt = jnp.min(dt, axis=1)
        it = jnp.argmin(dt, axis=1).astype(jnp.int32) + t * _TW
        win = mt < acc_v.astype(jnp.float32)
        acc_i = jnp.where(win, it, acc_i)
        acc_v = jnp.where(win, mt, acc_v.astype(jnp.float32)).astype(jnp.bfloat16)
    codes = acc_i

    onehot = (jax.lax.broadcasted_iota(jnp.int32, (_BT, _NUM_CODES), 1)
              == codes[:, None]).astype(jnp.float32)
    zq = jax.lax.dot_general(onehot, emb, (((1,), (0,)), ((), ())),
                             preferred_element_type=jnp.float32,
                             precision=jax.lax.Precision.HIGHEST)
    zq_ref[...] = z + (zq - z)
    codes_ref[...] = codes[:, None]

    @pl.when(i == 0)
    def _init():
        loss_ref[...] = jnp.zeros_like(loss_ref)
        avg_ref[...] = jnp.zeros_like(avg_ref)

    loss_ref[...] += jnp.sum((z - zq) ** 2, keepdims=True)
    avg_ref[...] += jnp.sum(onehot, axis=0, keepdims=True)

    @pl.when(i == nb - 1)
    def _finish():
        p = avg_ref[...] * (1.0 / (nb * _BT))
        neg_ent = jnp.sum(p * jnp.log(p + 1e-10), keepdims=True)
        ent_ref[...] = -neg_ent
        nent_ref[...] = -neg_ent / np.log(_NUM_CODES)
        avg_ref[...] = p
        loss_ref[...] *= 1.0 / (nb * _BT * _CODE_DIM)


def kernel(z, embed):
    orig_shape = z.shape
    flat_z = z.reshape(-1, _CODE_DIM)
    n_tok = flat_z.shape[0]
    nb = n_tok // _BT
    f32 = jnp.float32
    outs = pl.pallas_call(
        _vq_body,
        grid=(nb,),
        in_specs=[
            pl.BlockSpec((_BT, _CODE_DIM), lambda i: (i, 0)),
            pl.BlockSpec((_NUM_CODES, _CODE_DIM), lambda i: (0, 0)),
        ],
        out_specs=[
            pl.BlockSpec((_BT, _CODE_DIM), lambda i: (i, 0)),
            pl.BlockSpec((_BT, 1), lambda i: (i, 0)),
            pl.BlockSpec((1, 1), lambda i: (0, 0)),
            pl.BlockSpec((1, _NUM_CODES), lambda i: (0, 0)),
            pl.BlockSpec((1, 1), lambda i: (0, 0)),
            pl.BlockSpec((1, 1), lambda i: (0, 0)),
        ],
        out_shape=[
            jax.ShapeDtypeStruct((n_tok, _CODE_DIM), f32),
            jax.ShapeDtypeStruct((n_tok, 1), jnp.int32),
            jax.ShapeDtypeStruct((1, 1), f32),
            jax.ShapeDtypeStruct((1, _NUM_CODES), f32),
            jax.ShapeDtypeStruct((1, 1), f32),
            jax.ShapeDtypeStruct((1, 1), f32),
        ],
    )(flat_z, embed)
    zq_flat, codes_col, loss, avg, ent, nent = outs
    z_q = zq_flat.reshape(orig_shape)
    codes = codes_col.reshape(orig_shape[:-1])
    return (z_q, codes, loss.reshape(()), ent.reshape(()),
            nent.reshape(()), avg.reshape(_NUM_CODES))


# speedup vs baseline: 2.8589x; 2.8589x over previous
import functools

import jax
import jax.numpy as jnp
import numpy as np
from jax import lax
from jax.experimental import pallas as pl
from jax.experimental.pallas import tpu as pltpu
from jax.experimental.pallas import tpu_sc as plsc

_NC = 8192
_CD = 32
_NT = 8192
_BT = 256
_TW = 2048
_SC_CORES = 2
_SC_SUB = 16
_NW = _SC_CORES * _SC_SUB
_BPW = _NT // _NW
_CW = 128


def _argmin_body(z_ref, emb_ref, codes_ref, e2_ref, ebf_ref):
    i = pl.program_id(0)

    @pl.when(i == 0)
    def _prep():
        emb = emb_ref[...]
        e2_ref[...] = jnp.sum(emb * emb, axis=1, keepdims=True).T
        ebf_ref[...] = (-2.0 * emb).astype(jnp.bfloat16)

    z = z_ref[...]
    z2 = jnp.sum(z * z, axis=1, keepdims=True)
    m2 = jax.lax.dot_general(z.astype(jnp.bfloat16), ebf_ref[...],
                             (((1,), (1,)), ((), ())),
                             preferred_element_type=jnp.float32)
    dist = (z2 + m2) + e2_ref[...]

    acc_v = jnp.full((_BT,), jnp.inf, jnp.bfloat16)
    acc_i = jnp.zeros((_BT,), jnp.int32)
    for t in range(_NC // _TW):
        dt = dist[:, t * _TW:(t + 1) * _TW]
        mt = jnp.min(dt, axis=1)
        it = jnp.argmin(dt, axis=1).astype(jnp.int32) + t * _TW
        win = mt < acc_v.astype(jnp.float32)
        acc_i = jnp.where(win, it, acc_i)
        acc_v = jnp.where(win, mt, acc_v.astype(jnp.float32)).astype(jnp.bfloat16)
    codes_ref[...] = acc_i[:, None]


def _run_argmin(flat_z, embed):
    nb = _NT // _BT
    return pl.pallas_call(
        _argmin_body,
        grid=(nb,),
        in_specs=[
            pl.BlockSpec((_BT, _CD), lambda i: (i, 0)),
            pl.BlockSpec((_NC, _CD), lambda i: (0, 0)),
        ],
        out_specs=pl.BlockSpec((_BT, 1), lambda i: (i, 0)),
        out_shape=jax.ShapeDtypeStruct((_NT, 1), jnp.int32),
        scratch_shapes=[
            pltpu.VMEM((1, _NC), jnp.float32),
            pltpu.VMEM((_NC, _CD), jnp.bfloat16),
        ],
    )(flat_z, embed)


def _sc_body(codes2_hbm, emb_hbm, zeros_hbm, ones_hbm,
             zq_hbm, cnt_hbm, idx_v, rows_v, ones_v, shared, sem):
    c = lax.axis_index("c")
    s = lax.axis_index("s")
    wid = s * _SC_CORES + c
    base = wid * _BPW

    pltpu.sync_copy(zeros_hbm.at[pl.ds(s * 512, 512)],
                    shared.at[pl.ds(s * 512, 512)])
    pltpu.sync_copy(codes2_hbm.at[pl.ds(wid * 2, 2)], idx_v)
    pltpu.sync_copy(ones_hbm, ones_v)
    plsc.subcore_barrier()

    for j in range(2):
        pltpu.async_copy(emb_hbm.at[idx_v.at[j]],
                         rows_v.at[pl.ds(j * 128, 128)], sem).wait()
        pltpu.sync_copy(ones_v, shared.at[idx_v.at[j]], add=True)
    pltpu.sync_copy(rows_v, zq_hbm.at[pl.ds(base, _BPW)])

    plsc.subcore_barrier()
    pltpu.sync_copy(shared.at[pl.ds(s * 512, 512)],
                    cnt_hbm.at[pl.ds(c * _NC + s * 512, 512)])


@functools.partial(
    pl.kernel,
    out_type=[
        jax.ShapeDtypeStruct((_NT, 128), jnp.float32),
        jax.ShapeDtypeStruct((2 * _NC, _CW), jnp.float32),
    ],
    mesh=plsc.VectorSubcoreMesh(core_axis_name="c", subcore_axis_name="s",
                                num_cores=_SC_CORES),
    scratch_types=[
        pltpu.VMEM((2, 128), jnp.int32),
        pltpu.VMEM((_BPW, 128), jnp.float32),
        pltpu.VMEM((128, _CW), jnp.float32),
        pltpu.VMEM_SHARED((_NC, _CW), jnp.float32),
        pltpu.SemaphoreType.DMA,
    ],
)
def _sc_gather_count(codes2_hbm, emb_hbm, zeros_hbm, ones_hbm,
                     zq_hbm, cnt_hbm, idx_v, rows_v, ones_v, shared, sem):
    _sc_body(codes2_hbm, emb_hbm, zeros_hbm, ones_hbm,
             zq_hbm, cnt_hbm, idx_v, rows_v, ones_v, shared, sem)


def _fin_body(z_ref, zq_ref, cnt_ref, out_ref, loss_ref, ent_ref,
              nent_ref, avg_ref):
    z = z_ref[...]
    q = zq_ref[:, 0:_CD]
    out_ref[...] = z + (q - z)
    loss_ref[...] = jnp.sum((z - q) ** 2, keepdims=True) * (1.0 / (_NT * _CD))
    counts = cnt_ref[0:_NC, 0:1] + cnt_ref[_NC:2 * _NC, 0:1]
    p = counts * (1.0 / _NT)
    neg_ent = jnp.sum(p * jnp.log(p + 1e-10), keepdims=True)
    ent_ref[...] = -neg_ent
    nent_ref[...] = -neg_ent / np.log(_NC)
    avg_ref[...] = p


def _run_finalize(flat_z, zq_rows, cnt):
    f32 = jnp.float32
    return pl.pallas_call(
        _fin_body,
        out_shape=[
            jax.ShapeDtypeStruct((_NT, _CD), f32),
            jax.ShapeDtypeStruct((1, 1), f32),
            jax.ShapeDtypeStruct((1, 1), f32),
            jax.ShapeDtypeStruct((1, 1), f32),
            jax.ShapeDtypeStruct((_NC, 1), f32),
        ],
    )(flat_z, zq_rows, cnt)


def kernel(z, embed):
    orig_shape = z.shape
    flat_z = z.reshape(-1, _CD)
    codes_col = _run_argmin(flat_z, embed)
    codes2 = codes_col.reshape(_NW * 2, 128)
    emb_pad = jnp.pad(embed, ((0, 0), (0, 128 - _CD)))
    zeros16 = jnp.zeros((_NC, _CW), jnp.float32)
    ones16 = jnp.ones((128, _CW), jnp.float32)
    zq_rows, cnt = _sc_gather_count(codes2, emb_pad, zeros16, ones16)
    zq_ste, loss, ent, nent, avg = _run_finalize(flat_z, zq_rows, cnt)
    return (zq_ste.reshape(orig_shape), codes_col.reshape(orig_shape[:-1]),
            loss.reshape(()), ent.reshape(()), nent.reshape(()),
            avg.reshape(_NC))
